# baseline (speedup 1.0000x reference)
import functools
import numpy as np
import jax
import jax.numpy as jnp
from jax import lax
from jax.experimental import pallas as pl
from jax.experimental.pallas import tpu as pltpu
from jax.experimental.pallas import tpu_sc as plsc

H = 16
OMEGA = 30.0
BE = 3200
CH = 128
NC = 2
NS = 16


def _mesh():
    return plsc.VectorSubcoreMesh(core_axis_name="c", subcore_axis_name="s")


_SC_PARAMS = pltpu.CompilerParams(needs_layout_passes=False,
                                  use_tc_tiling_on_sc=False)


def _sc_gather(u_emb, i_emb, u_t, i_t, uidx, iidx, et):
    E = et.shape[0]
    N_u = u_t.shape[0]
    N_i = i_t.shape[0]
    nch = E // CH
    nw = NC * NS
    kmax = (nch + nw - 1) // nw
    f32 = jnp.float32

    @functools.partial(
        pl.kernel,
        out_type=[
            jax.ShapeDtypeStruct((nch, CH, H), f32),
            jax.ShapeDtypeStruct((nch, CH, H), f32),
            jax.ShapeDtypeStruct((E,), f32),
            jax.ShapeDtypeStruct((E,), f32),
        ],
        mesh=_mesh(),
        scratch_types=[
            pltpu.VMEM((N_u,), f32),
            pltpu.VMEM((N_i,), f32),
            pltpu.VMEM((CH,), jnp.int32),
            pltpu.VMEM((CH,), jnp.int32),
            pltpu.VMEM((CH, H), f32),
            pltpu.VMEM((CH, H), f32),
            pltpu.VMEM((CH,), f32),
            pltpu.VMEM((CH,), f32),
            pltpu.VMEM((CH,), f32),
            pltpu.SemaphoreType.DMA,
            pltpu.SemaphoreType.DMA,
        ],
        compiler_params=_SC_PARAMS,
    )
    def k(u_emb_h, i_emb_h, u_t_h, i_t_h, uidx_h, iidx_h, et_h,
          ue_c_h, ie_c_h, rl_u_h, rl_i_h,
          ut_tab, it_tab, idx_u, idx_i, rows_u, rows_i, et_v, ru_v, ri_v,
          sem_u, sem_i):
        wid = lax.axis_index("s") * NC + lax.axis_index("c")
        pltpu.sync_copy(u_t_h, ut_tab)
        pltpu.sync_copy(i_t_h, it_tab)

        def chunk(kk, carry):
            c = kk * nw + wid

            @pl.when(c < nch)
            def _():
                sl = pl.ds(c * CH, CH)
                pltpu.sync_copy(uidx_h.at[sl], idx_u)
                pltpu.sync_copy(iidx_h.at[sl], idx_i)
                pltpu.sync_copy(et_h.at[sl], et_v)
                cp_u = pltpu.async_copy(u_emb_h.at[idx_u], rows_u, sem_u)
                cp_i = pltpu.async_copy(i_emb_h.at[idx_i], rows_i, sem_i)
                for v in range(CH // 16):
                    vs = pl.ds(16 * v, 16)
                    ev = et_v[vs]
                    ru_v[vs] = plsc.load_gather(ut_tab, [idx_u[vs]]) - ev
                    ri_v[vs] = plsc.load_gather(it_tab, [idx_i[vs]]) - ev
                cp_u.wait()
                cp_i.wait()
                pltpu.sync_copy(rows_u, ue_c_h.at[c])
                pltpu.sync_copy(rows_i, ie_c_h.at[c])
                pltpu.sync_copy(ru_v, rl_u_h.at[sl])
                pltpu.sync_copy(ri_v, rl_i_h.at[sl])

            return carry

        lax.fori_loop(0, kmax, chunk, 0)

    return k(u_emb, i_emb, u_t, i_t, uidx, iidx, et)


def _fast_sin2(z):
    u = z - jnp.round(z)
    u2 = u * u
    c1 = jnp.float32(6.2830887)
    c3 = jnp.float32(-41.333252)
    c5 = jnp.float32(81.40014)
    c7 = jnp.float32(-74.67622)
    c9 = jnp.float32(33.16881)
    return u * (c1 + u2 * (c3 + u2 * (c5 + u2 * (c7 + u2 * c9))))


def _dense_body(rlu_ref, rli_ref, ue_ref, ie_ref,
                wu1_ref, wu2_ref, wu3_ref, wi1_ref, wi2_ref, wi3_ref,
                tb_ref, sb_ref, out_ref):
    f32 = jnp.float32
    bf16 = jnp.bfloat16
    tb = tb_ref[...]
    sb = sb_ref[...]

    def side(rel16, w1t, w2b, w3b, emb):
        x = _fast_sin2(rel16 * w1t)
        x = _fast_sin2(jnp.dot(x, w2b, preferred_element_type=f32))
        y = jnp.dot(x.astype(bf16), w3b, preferred_element_type=f32).astype(bf16)
        rep = jnp.dot(emb.astype(bf16), tb, preferred_element_type=f32).astype(bf16)
        return jnp.dot(y * rep, sb, preferred_element_type=f32)

    out_ref[0, :, :] = side(rli_ref[...], wi1_ref[...], wi2_ref[...],
                            wi3_ref[...], ie_ref[...])
    out_ref[1, :, :] = side(rlu_ref[...], wu1_ref[...], wu2_ref[...],
                            wu3_ref[...], ue_ref[...])


def _tc_dense(rl_u, rl_i, ue_p, ie_p, Wu1, Wu2, Wu3, Wi1, Wi2, Wi3):
    R = rl_u.shape[0]
    rb = BE // 8
    nb = R // rb
    f32 = jnp.float32
    bf16 = jnp.bfloat16
    q = jnp.float32(OMEGA / (2.0 * np.pi))
    eye8 = np.eye(8, dtype=np.float32)

    def w1tile(w1):
        return jnp.tile((w1 * q).reshape(H), 8).reshape(1, 8 * H)

    def blockdiag(w):
        return jnp.kron(jnp.asarray(eye8), w)

    W2Bu = blockdiag(Wu2 * q)
    W2Bi = blockdiag(Wi2 * q)
    W3Bu = blockdiag(Wu3).astype(bf16)
    W3Bi = blockdiag(Wi3).astype(bf16)
    T16 = np.tile(np.eye(H, dtype=np.float32), (1, H))
    TB = jnp.asarray(np.kron(eye8, T16)).astype(bf16)
    S256 = np.kron(np.eye(H, dtype=np.float32), np.ones((H, 1), np.float32))
    SB = jnp.asarray(np.kron(eye8, S256)).astype(bf16)

    def full(shape):
        return pl.BlockSpec(shape, lambda b: (0,) * len(shape))

    call = pl.pallas_call(
        _dense_body,
        grid=(nb,),
        in_specs=[
            pl.BlockSpec((rb, 8 * H), lambda b: (b, 0)),
            pl.BlockSpec((rb, 8 * H), lambda b: (b, 0)),
            pl.BlockSpec((rb, 8 * H), lambda b: (b, 0)),
            pl.BlockSpec((rb, 8 * H), lambda b: (b, 0)),
            full((1, 8 * H)), full((8 * H, 8 * H)), full((8 * H, 8 * H * H)),
            full((1, 8 * H)), full((8 * H, 8 * H)), full((8 * H, 8 * H * H)),
            full((8 * H, 8 * H * H)), full((8 * H * H, 8 * H)),
        ],
        out_specs=pl.BlockSpec((2, rb, 8 * H), lambda b: (0, b, 0)),
        out_shape=jax.ShapeDtypeStruct((2, R, 8 * H), f32),
    )
    return call(rl_u, rl_i, ue_p, ie_p,
                w1tile(Wu1), W2Bu, W3Bu,
                w1tile(Wi1), W2Bi, W3Bi, TB, SB)


def _sc_scatter(msgs, idxs, N):
    E = idxs.shape[1]
    nch = E // CH
    kmax = (nch + NS - 1) // NS
    rows = N // NS
    f32 = jnp.float32

    @functools.partial(
        pl.kernel,
        out_type=jax.ShapeDtypeStruct((2, N, H), f32),
        mesh=_mesh(),
        scratch_types=[
            pltpu.VMEM((CH, H), f32),
            pltpu.VMEM((CH,), jnp.int32),
            pltpu.VMEM((rows, H), f32),
            pltpu.VMEM_SHARED((N, H), f32),
        ],
        compiler_params=_SC_PARAMS,
    )
    def k(msgs_h, idxs_h, out_h, msg_v, idx_v, slice_v, acc):
        cid = lax.axis_index("c")
        sid = lax.axis_index("s")

        def zrow(j, carry):
            slice_v[j, :] = jnp.zeros((H,), f32)
            return carry

        lax.fori_loop(0, rows, zrow, 0)
        pltpu.sync_copy(slice_v, acc.at[pl.ds(sid * rows, rows)])
        plsc.subcore_barrier()

        def chunk(kk, carry):
            c = kk * NS + sid

            @pl.when(c < nch)
            def _():
                pltpu.sync_copy(idxs_h.at[cid, pl.ds(c * CH, CH)], idx_v)
                pltpu.sync_copy(msgs_h.at[cid, c], msg_v)
                pltpu.sync_copy(msg_v, acc.at[idx_v], add=True)

            return carry

        lax.fori_loop(0, kmax, chunk, 0)
        plsc.subcore_barrier()

        osl = pl.ds(sid * rows, rows)
        pltpu.sync_copy(acc.at[osl], slice_v)
        pltpu.sync_copy(slice_v, out_h.at[cid, osl])

    return k(msgs, idxs)


def kernel(u_embedded, i_embedded, user_per_trans, item_per_trans, edges_t,
           u_t, i_t, Wu1, Wu2, Wu3, Wi1, Wi2, Wi3):
    E = edges_t.shape[0]
    N = u_embedded.shape[0]
    uidx = user_per_trans.astype(jnp.int32)
    iidx = item_per_trans.astype(jnp.int32)
    ue_c, ie_c, rl_u, rl_i = _sc_gather(
        u_embedded, i_embedded, u_t, i_t, uidx, iidx, edges_t)
    R = E // 8
    rl_u16 = jnp.repeat(rl_u[:, None], H, 1).reshape(R, 8 * H)
    rl_i16 = jnp.repeat(rl_i[:, None], H, 1).reshape(R, 8 * H)
    msgs = _tc_dense(rl_u16, rl_i16,
                     ue_c.reshape(R, 8 * H), ie_c.reshape(R, 8 * H),
                     Wu1, Wu2, Wu3, Wi1, Wi2, Wi3)
    msgs4 = msgs.reshape(2, E // CH, CH, H)
    idxs = jnp.stack([uidx, iidx])
    out = _sc_scatter(msgs4, idxs, N)
    return (out[0], out[1])


# speedup vs baseline: 1.3384x; 1.3384x over previous
import functools
import numpy as np
import jax
import jax.numpy as jnp
from jax import lax
from jax.experimental import pallas as pl
from jax.experimental.pallas import tpu as pltpu
from jax.experimental.pallas import tpu_sc as plsc

H = 16
OMEGA = 30.0
BE = 3200
CH = 128
NC = 2
NS = 16


def _mesh():
    return plsc.VectorSubcoreMesh(core_axis_name="c", subcore_axis_name="s")


_SC_PARAMS = pltpu.CompilerParams(needs_layout_passes=False,
                                  use_tc_tiling_on_sc=False)


def _sc_gather(u_emb, i_emb, u_t, i_t, uidx, iidx, et):
    E = et.shape[0]
    N_u = u_t.shape[0]
    N_i = i_t.shape[0]
    nch = E // CH
    nw = NC * NS
    kmax = (nch + nw - 1) // nw
    f32 = jnp.float32

    @functools.partial(
        pl.kernel,
        out_type=[
            jax.ShapeDtypeStruct((nch, CH, H), f32),
            jax.ShapeDtypeStruct((nch, CH, H), f32),
            jax.ShapeDtypeStruct((E,), f32),
            jax.ShapeDtypeStruct((E,), f32),
        ],
        mesh=_mesh(),
        scratch_types=[
            pltpu.VMEM((N_u,), f32),
            pltpu.VMEM((N_i,), f32),
            pltpu.VMEM((CH,), jnp.int32),
            pltpu.VMEM((CH,), jnp.int32),
            pltpu.VMEM((CH, H), f32),
            pltpu.VMEM((CH, H), f32),
            pltpu.VMEM((CH,), f32),
            pltpu.VMEM((CH,), f32),
            pltpu.VMEM((CH,), f32),
            pltpu.SemaphoreType.DMA,
            pltpu.SemaphoreType.DMA,
        ],
        compiler_params=_SC_PARAMS,
    )
    def k(u_emb_h, i_emb_h, u_t_h, i_t_h, uidx_h, iidx_h, et_h,
          ue_c_h, ie_c_h, rl_u_h, rl_i_h,
          ut_tab, it_tab, idx_u, idx_i, rows_u, rows_i, et_v, ru_v, ri_v,
          sem_u, sem_i):
        wid = lax.axis_index("s") * NC + lax.axis_index("c")
        pltpu.sync_copy(u_t_h, ut_tab)
        pltpu.sync_copy(i_t_h, it_tab)

        def chunk(kk, carry):
            c = kk * nw + wid

            @pl.when(c < nch)
            def _():
                sl = pl.ds(c * CH, CH)
                pltpu.sync_copy(uidx_h.at[sl], idx_u)
                pltpu.sync_copy(iidx_h.at[sl], idx_i)
                pltpu.sync_copy(et_h.at[sl], et_v)
                cp_u = pltpu.async_copy(u_emb_h.at[idx_u], rows_u, sem_u)
                cp_i = pltpu.async_copy(i_emb_h.at[idx_i], rows_i, sem_i)
                for v in range(CH // 16):
                    vs = pl.ds(16 * v, 16)
                    ev = et_v[vs]
                    ru_v[vs] = plsc.load_gather(ut_tab, [idx_u[vs]]) - ev
                    ri_v[vs] = plsc.load_gather(it_tab, [idx_i[vs]]) - ev
                cp_u.wait()
                cp_i.wait()
                pltpu.sync_copy(rows_u, ue_c_h.at[c])
                pltpu.sync_copy(rows_i, ie_c_h.at[c])
                pltpu.sync_copy(ru_v, rl_u_h.at[sl])
                pltpu.sync_copy(ri_v, rl_i_h.at[sl])

            return carry

        lax.fori_loop(0, kmax, chunk, 0)

    return k(u_emb, i_emb, u_t, i_t, uidx, iidx, et)


def _fast_sin2(z):
    u = z - jnp.round(z)
    u2 = u * u
    c1 = jnp.float32(6.2830887)
    c3 = jnp.float32(-41.333252)
    c5 = jnp.float32(81.40014)
    c7 = jnp.float32(-74.67622)
    c9 = jnp.float32(33.16881)
    return u * (c1 + u2 * (c3 + u2 * (c5 + u2 * (c7 + u2 * c9))))


def _dense_body(rlu_ref, rli_ref, ue_ref, ie_ref,
                wu1_ref, wu2_ref, wu3_ref, wi1_ref, wi2_ref, wi3_ref,
                tb_ref, sb_ref, out_ref):
    f32 = jnp.float32
    bf16 = jnp.bfloat16
    tb = tb_ref[...]
    sb = sb_ref[...]

    def side(rel16, w1t, w2b, w3b, emb):
        x = _fast_sin2(rel16 * w1t)
        x = _fast_sin2(jnp.dot(x, w2b, preferred_element_type=f32))
        y = jnp.dot(x.astype(bf16), w3b, preferred_element_type=f32).astype(bf16)
        rep = jnp.dot(emb.astype(bf16), tb, preferred_element_type=f32).astype(bf16)
        return jnp.dot(y * rep, sb, preferred_element_type=f32)

    out_ref[0, :, :] = side(rli_ref[...], wi1_ref[...], wi2_ref[...],
                            wi3_ref[...], ie_ref[...])
    out_ref[1, :, :] = side(rlu_ref[...], wu1_ref[...], wu2_ref[...],
                            wu3_ref[...], ue_ref[...])


def _tc_dense(rl_u, rl_i, ue_p, ie_p, Wu1, Wu2, Wu3, Wi1, Wi2, Wi3):
    R = rl_u.shape[0]
    rb = BE // 8
    nb = R // rb
    f32 = jnp.float32
    bf16 = jnp.bfloat16
    q = jnp.float32(OMEGA / (2.0 * np.pi))
    eye8 = np.eye(8, dtype=np.float32)

    def w1tile(w1):
        return jnp.tile((w1 * q).reshape(H), 8).reshape(1, 8 * H)

    def blockdiag(w):
        return jnp.kron(jnp.asarray(eye8), w)

    W2Bu = blockdiag(Wu2 * q)
    W2Bi = blockdiag(Wi2 * q)
    W3Bu = blockdiag(Wu3).astype(bf16)
    W3Bi = blockdiag(Wi3).astype(bf16)
    T16 = np.tile(np.eye(H, dtype=np.float32), (1, H))
    TB = jnp.asarray(np.kron(eye8, T16)).astype(bf16)
    S256 = np.kron(np.eye(H, dtype=np.float32), np.ones((H, 1), np.float32))
    SB = jnp.asarray(np.kron(eye8, S256)).astype(bf16)

    def full(shape):
        return pl.BlockSpec(shape, lambda b: (0,) * len(shape))

    call = pl.pallas_call(
        _dense_body,
        grid=(nb,),
        in_specs=[
            pl.BlockSpec((rb, 8 * H), lambda b: (b, 0)),
            pl.BlockSpec((rb, 8 * H), lambda b: (b, 0)),
            pl.BlockSpec((rb, 8 * H), lambda b: (b, 0)),
            pl.BlockSpec((rb, 8 * H), lambda b: (b, 0)),
            full((1, 8 * H)), full((8 * H, 8 * H)), full((8 * H, 8 * H * H)),
            full((1, 8 * H)), full((8 * H, 8 * H)), full((8 * H, 8 * H * H)),
            full((8 * H, 8 * H * H)), full((8 * H * H, 8 * H)),
        ],
        out_specs=pl.BlockSpec((2, rb, 8 * H), lambda b: (0, b, 0)),
        out_shape=jax.ShapeDtypeStruct((2, R, 8 * H), f32),
    )
    return call(rl_u, rl_i, ue_p, ie_p,
                w1tile(Wu1), W2Bu, W3Bu,
                w1tile(Wi1), W2Bi, W3Bi, TB, SB)


def _sc_scatter(msgs, idxs, N):
    E = idxs.shape[1]
    nch = E // CH
    kmax = (nch + NS - 1) // NS
    rows = N // NS
    f32 = jnp.float32

    @functools.partial(
        pl.kernel,
        out_type=jax.ShapeDtypeStruct((2, N, H), f32),
        mesh=_mesh(),
        scratch_types=[
            pltpu.VMEM((CH, H), f32),
            pltpu.VMEM((CH,), jnp.int32),
            pltpu.VMEM((rows, H), f32),
            pltpu.VMEM_SHARED((N, H), f32),
        ],
        compiler_params=_SC_PARAMS,
    )
    def k(msgs_h, idxs_h, out_h, msg_v, idx_v, slice_v, acc):
        cid = lax.axis_index("c")
        sid = lax.axis_index("s")

        def zrow(j, carry):
            slice_v[j, :] = jnp.zeros((H,), f32)
            return carry

        lax.fori_loop(0, rows, zrow, 0)
        pltpu.sync_copy(slice_v, acc.at[pl.ds(sid * rows, rows)])
        plsc.subcore_barrier()

        def chunk(kk, carry):
            c = kk * NS + sid

            @pl.when(c < nch)
            def _():
                pltpu.sync_copy(idxs_h.at[cid, pl.ds(c * CH, CH)], idx_v)
                pltpu.sync_copy(msgs_h.at[cid, c], msg_v)
                pltpu.sync_copy(msg_v, acc.at[idx_v], add=True)

            return carry

        lax.fori_loop(0, kmax, chunk, 0)
        plsc.subcore_barrier()

        osl = pl.ds(sid * rows, rows)
        pltpu.sync_copy(acc.at[osl], slice_v)
        pltpu.sync_copy(slice_v, out_h.at[cid, osl])

    return k(msgs, idxs)


def kernel(u_embedded, i_embedded, user_per_trans, item_per_trans, edges_t,
           u_t, i_t, Wu1, Wu2, Wu3, Wi1, Wi2, Wi3):
    E = edges_t.shape[0]
    N = u_embedded.shape[0]
    uidx = user_per_trans.astype(jnp.int32)
    iidx = item_per_trans.astype(jnp.int32)
    ue_c, ie_c, rl_u, rl_i = _sc_gather(
        u_embedded, i_embedded, u_t, i_t, uidx, iidx, edges_t)
    R = E // 8
    rl_u16 = jnp.repeat(rl_u.reshape(R, 8), H, axis=1)
    rl_i16 = jnp.repeat(rl_i.reshape(R, 8), H, axis=1)
    msgs = _tc_dense(rl_u16, rl_i16,
                     ue_c.reshape(R, 8 * H), ie_c.reshape(R, 8 * H),
                     Wu1, Wu2, Wu3, Wi1, Wi2, Wi3)
    msgs4 = msgs.reshape(2, E // CH, CH, H)
    idxs = jnp.stack([uidx, iidx])
    out = _sc_scatter(msgs4, idxs, N)
    return (out[0], out[1])


# speedup vs baseline: 1.5695x; 1.1727x over previous
import functools
import numpy as np
import jax
import jax.numpy as jnp
from jax import lax
from jax.experimental import pallas as pl
from jax.experimental.pallas import tpu as pltpu
from jax.experimental.pallas import tpu_sc as plsc

H = 16
OMEGA = 30.0
BE = 3200
CH = 128
NC = 2
NS = 16


def _mesh():
    return plsc.VectorSubcoreMesh(core_axis_name="c", subcore_axis_name="s")


_SC_PARAMS = pltpu.CompilerParams(needs_layout_passes=False,
                                  use_tc_tiling_on_sc=False)


def _sc_gather(u_emb, i_emb, u_t, i_t, uidx, iidx, et):
    E = et.shape[0]
    N_u = u_t.shape[0]
    N_i = i_t.shape[0]
    nch = E // CH
    nw = NC * NS
    kmax = (nch + nw - 1) // nw
    f32 = jnp.float32

    @functools.partial(
        pl.kernel,
        out_type=[
            jax.ShapeDtypeStruct((nch, CH, H), f32),
            jax.ShapeDtypeStruct((nch, CH, H), f32),
            jax.ShapeDtypeStruct((E,), f32),
            jax.ShapeDtypeStruct((E,), f32),
        ],
        mesh=_mesh(),
        scratch_types=[
            pltpu.VMEM((N_u,), f32),
            pltpu.VMEM((N_i,), f32),
            pltpu.VMEM((CH,), jnp.int32),
            pltpu.VMEM((CH,), jnp.int32),
            pltpu.VMEM((CH, H), f32),
            pltpu.VMEM((CH, H), f32),
            pltpu.VMEM((CH,), f32),
            pltpu.VMEM((CH,), f32),
            pltpu.VMEM((CH,), f32),
            pltpu.SemaphoreType.DMA,
            pltpu.SemaphoreType.DMA,
        ],
        compiler_params=_SC_PARAMS,
    )
    def k(u_emb_h, i_emb_h, u_t_h, i_t_h, uidx_h, iidx_h, et_h,
          ue_c_h, ie_c_h, rl_u_h, rl_i_h,
          ut_tab, it_tab, idx_u, idx_i, rows_u, rows_i, et_v, ru_v, ri_v,
          sem_u, sem_i):
        wid = lax.axis_index("s") * NC + lax.axis_index("c")
        pltpu.sync_copy(u_t_h, ut_tab)
        pltpu.sync_copy(i_t_h, it_tab)

        def chunk(kk, carry):
            c = kk * nw + wid

            @pl.when(c < nch)
            def _():
                sl = pl.ds(c * CH, CH)
                c1 = pltpu.async_copy(uidx_h.at[sl], idx_u, sem_u)
                c2 = pltpu.async_copy(iidx_h.at[sl], idx_i, sem_u)
                c3 = pltpu.async_copy(et_h.at[sl], et_v, sem_u)
                c1.wait()
                c2.wait()
                c3.wait()
                cp_u = pltpu.async_copy(u_emb_h.at[idx_u], rows_u, sem_u)
                cp_i = pltpu.async_copy(i_emb_h.at[idx_i], rows_i, sem_i)
                for v in range(CH // 16):
                    vs = pl.ds(16 * v, 16)
                    ev = et_v[vs]
                    ru_v[vs] = plsc.load_gather(ut_tab, [idx_u[vs]]) - ev
                    ri_v[vs] = plsc.load_gather(it_tab, [idx_i[vs]]) - ev
                cp_u.wait()
                cp_i.wait()
                o1 = pltpu.async_copy(rows_u, ue_c_h.at[c], sem_u)
                o2 = pltpu.async_copy(rows_i, ie_c_h.at[c], sem_u)
                o3 = pltpu.async_copy(ru_v, rl_u_h.at[sl], sem_i)
                o4 = pltpu.async_copy(ri_v, rl_i_h.at[sl], sem_i)
                o1.wait()
                o2.wait()
                o3.wait()
                o4.wait()

            return carry

        lax.fori_loop(0, kmax, chunk, 0)

    return k(u_emb, i_emb, u_t, i_t, uidx, iidx, et)


def _fast_sin2(z):
    u = z - jnp.round(z)
    u2 = u * u
    c1 = jnp.float32(6.2830887)
    c3 = jnp.float32(-41.333252)
    c5 = jnp.float32(81.40014)
    c7 = jnp.float32(-74.67622)
    c9 = jnp.float32(33.16881)
    return u * (c1 + u2 * (c3 + u2 * (c5 + u2 * (c7 + u2 * c9))))


def _dense_body(rlu_ref, rli_ref, ue_ref, ie_ref,
                wu1_ref, wu2_ref, wu3_ref, wi1_ref, wi2_ref, wi3_ref,
                tb_ref, sb_ref, out_ref):
    f32 = jnp.float32
    bf16 = jnp.bfloat16
    tb = tb_ref[...]
    sb = sb_ref[...]

    def side(rel16, w1t, w2b, w3b, emb):
        x = _fast_sin2(rel16 * w1t)
        x = _fast_sin2(jnp.dot(x, w2b, preferred_element_type=f32))
        y = jnp.dot(x.astype(bf16), w3b, preferred_element_type=f32).astype(bf16)
        rep = jnp.dot(emb.astype(bf16), tb, preferred_element_type=f32).astype(bf16)
        return jnp.dot(y * rep, sb, preferred_element_type=f32)

    out_ref[0, :, :] = side(rli_ref[...], wi1_ref[...], wi2_ref[...],
                            wi3_ref[...], ie_ref[...])
    out_ref[1, :, :] = side(rlu_ref[...], wu1_ref[...], wu2_ref[...],
                            wu3_ref[...], ue_ref[...])


def _tc_dense(rl_u, rl_i, ue_p, ie_p, Wu1, Wu2, Wu3, Wi1, Wi2, Wi3):
    R = rl_u.shape[0]
    rb = BE // 8
    nb = R // rb
    f32 = jnp.float32
    bf16 = jnp.bfloat16
    q = jnp.float32(OMEGA / (2.0 * np.pi))
    eye8 = np.eye(8, dtype=np.float32)

    def w1tile(w1):
        return jnp.tile((w1 * q).reshape(H), 8).reshape(1, 8 * H)

    def blockdiag(w):
        return jnp.kron(jnp.asarray(eye8), w)

    W2Bu = blockdiag(Wu2 * q)
    W2Bi = blockdiag(Wi2 * q)
    W3Bu = blockdiag(Wu3).astype(bf16)
    W3Bi = blockdiag(Wi3).astype(bf16)
    T16 = np.tile(np.eye(H, dtype=np.float32), (1, H))
    TB = jnp.asarray(np.kron(eye8, T16)).astype(bf16)
    S256 = np.kron(np.eye(H, dtype=np.float32), np.ones((H, 1), np.float32))
    SB = jnp.asarray(np.kron(eye8, S256)).astype(bf16)

    def full(shape):
        return pl.BlockSpec(shape, lambda b: (0,) * len(shape))

    call = pl.pallas_call(
        _dense_body,
        grid=(nb,),
        in_specs=[
            pl.BlockSpec((rb, 8 * H), lambda b: (b, 0)),
            pl.BlockSpec((rb, 8 * H), lambda b: (b, 0)),
            pl.BlockSpec((rb, 8 * H), lambda b: (b, 0)),
            pl.BlockSpec((rb, 8 * H), lambda b: (b, 0)),
            full((1, 8 * H)), full((8 * H, 8 * H)), full((8 * H, 8 * H * H)),
            full((1, 8 * H)), full((8 * H, 8 * H)), full((8 * H, 8 * H * H)),
            full((8 * H, 8 * H * H)), full((8 * H * H, 8 * H)),
        ],
        out_specs=pl.BlockSpec((2, rb, 8 * H), lambda b: (0, b, 0)),
        out_shape=jax.ShapeDtypeStruct((2, R, 8 * H), f32),
    )
    return call(rl_u, rl_i, ue_p, ie_p,
                w1tile(Wu1), W2Bu, W3Bu,
                w1tile(Wi1), W2Bi, W3Bi, TB, SB)


def _sc_scatter(msgs, idxs, N):
    E = idxs.shape[1]
    nch = E // CH
    kmax = (nch + NS - 1) // NS
    rows = N // NS
    f32 = jnp.float32

    @functools.partial(
        pl.kernel,
        out_type=jax.ShapeDtypeStruct((2, N, H), f32),
        mesh=_mesh(),
        scratch_types=[
            pltpu.VMEM((CH, H), f32),
            pltpu.VMEM((CH,), jnp.int32),
            pltpu.VMEM((rows, H), f32),
            pltpu.VMEM_SHARED((N, H), f32),
            pltpu.SemaphoreType.DMA,
        ],
        compiler_params=_SC_PARAMS,
    )
    def k(msgs_h, idxs_h, out_h, msg_v, idx_v, slice_v, acc, sem):
        cid = lax.axis_index("c")
        sid = lax.axis_index("s")

        def zrow(j, carry):
            slice_v[j, :] = jnp.zeros((H,), f32)
            return carry

        lax.fori_loop(0, rows, zrow, 0)
        pltpu.sync_copy(slice_v, acc.at[pl.ds(sid * rows, rows)])
        plsc.subcore_barrier()

        def chunk(kk, carry):
            c = kk * NS + sid

            @pl.when(c < nch)
            def _():
                c1 = pltpu.async_copy(idxs_h.at[cid, pl.ds(c * CH, CH)], idx_v, sem)
                c2 = pltpu.async_copy(msgs_h.at[cid, c], msg_v, sem)
                c1.wait()
                c2.wait()
                pltpu.sync_copy(msg_v, acc.at[idx_v], add=True)

            return carry

        lax.fori_loop(0, kmax, chunk, 0)
        plsc.subcore_barrier()

        osl = pl.ds(sid * rows, rows)
        pltpu.sync_copy(acc.at[osl], slice_v)
        pltpu.sync_copy(slice_v, out_h.at[cid, osl])

    return k(msgs, idxs)


def kernel(u_embedded, i_embedded, user_per_trans, item_per_trans, edges_t,
           u_t, i_t, Wu1, Wu2, Wu3, Wi1, Wi2, Wi3):
    E = edges_t.shape[0]
    N = u_embedded.shape[0]
    uidx = user_per_trans.astype(jnp.int32)
    iidx = item_per_trans.astype(jnp.int32)
    ue_c, ie_c, rl_u, rl_i = _sc_gather(
        u_embedded, i_embedded, u_t, i_t, uidx, iidx, edges_t)
    R = E // 8
    rl_u16 = jnp.repeat(rl_u.reshape(R, 8), H, axis=1)
    rl_i16 = jnp.repeat(rl_i.reshape(R, 8), H, axis=1)
    msgs = _tc_dense(rl_u16, rl_i16,
                     ue_c.reshape(R, 8 * H), ie_c.reshape(R, 8 * H),
                     Wu1, Wu2, Wu3, Wi1, Wi2, Wi3)
    msgs4 = msgs.reshape(2, E // CH, CH, H)
    idxs = jnp.stack([uidx, iidx])
    out = _sc_scatter(msgs4, idxs, N)
    return (out[0], out[1])
